# baseline (speedup 1.0000x reference)
import functools

import jax
import jax.numpy as jnp
from jax import lax
from jax.experimental import pallas as pl
from jax.experimental.pallas import tpu as pltpu
from jax.experimental.pallas import tpu_sc as plsc

_NC, _NS, _L = 2, 16, 16
_NW = _NC * _NS

_SC_PARAMS = pltpu.CompilerParams(
    needs_layout_passes=False, use_tc_tiling_on_sc=False)


def _mesh():
    return plsc.VectorSubcoreMesh(core_axis_name="c", subcore_axis_name="s")


@functools.cache
def _builder(C, N, HW, W, KB):
    PW = N // _NW
    NB = PW // KB
    G = KB // _L
    SPAN = KB + W + 8

    @functools.partial(
        pl.kernel,
        out_type=jax.ShapeDtypeStruct((N, 16), jnp.float32),
        mesh=_mesh(),
        compiler_params=_SC_PARAMS,
        scratch_types=[
            pltpu.VMEM((2, 4, SPAN), jnp.float32),
            pltpu.VMEM((2, KB, _L), jnp.float32),
            pltpu.SemaphoreType.DMA,
            pltpu.SemaphoreType.DMA,
        ],
    )
    def body(vf_hbm, t_hbm, streams_v, rows_v, sem_i, sem_o):
        wid = lax.axis_index("s") * _NC + lax.axis_index("c")
        base_w = wid * PW
        iota = lax.iota(jnp.int32, _L)

        def in_copies(b, p):
            base = base_w + b * KB
            return [
                pltpu.make_async_copy(
                    vf_hbm.at[pl.ds(c * N + base + zo, SPAN)],
                    streams_v.at[p, c * 2 + zi],
                    sem_i,
                )
                for c in range(C)
                for zi, zo in enumerate((0, HW))
            ]

        def out_copy(b, p):
            base = base_w + b * KB
            return pltpu.make_async_copy(
                rows_v.at[p], t_hbm.at[pl.ds(base, KB)], sem_o)

        for cpy in in_copies(0, 0):
            cpy.start()

        @pl.loop(0, NB, step=2)
        def _blk2(b0):
            for p in range(2):
                b = b0 + p
                for cpy in in_copies(b, p):
                    cpy.wait()

                @pl.when(b + 1 < NB)
                def _():
                    for cpy in in_copies(b + 1, 1 - p):
                        cpy.start()

                @pl.when(b >= 2)
                def _():
                    out_copy(b - 2, p).wait()

                @pl.loop(0, G, unroll=4)
                def _ilv(g):
                    s = g * _L
                    ridx = iota + s
                    for c in range(C):
                        for zi in range(2):
                            for yx, yxo in enumerate((0, 1, W, W + 1)):
                                col = c * 8 + zi * 4 + yx
                                vec = streams_v[p, c * 2 + zi,
                                                pl.ds(s + yxo, _L)]
                                plsc.store_scatter(
                                    rows_v.at[p],
                                    [ridx, jnp.full((_L,), col, jnp.int32)],
                                    vec,
                                )

                out_copy(b, p).start()

        out_copy(NB - 2, 0).wait()
        out_copy(NB - 1, 1).wait()

    return body


@functools.cache
def _sampler(D, H, W, C, N, K):
    PW = N // _NW
    NB = PW // K
    G = K // _L
    CH = 128
    NCH = K // CH

    @functools.partial(
        pl.kernel,
        out_type=jax.ShapeDtypeStruct((C * N,), jnp.float32),
        mesh=_mesh(),
        compiler_params=_SC_PARAMS,
        scratch_types=[
            pltpu.VMEM((2, 3, K), jnp.float32),
            pltpu.VMEM((2, K), jnp.int32),
            pltpu.VMEM((2, 3, K), jnp.float32),
            pltpu.VMEM((2, K, _L), jnp.float32),
            pltpu.VMEM((2, C, K), jnp.float32),
            pltpu.SemaphoreType.DMA,
            pltpu.SemaphoreType.DMA,
            pltpu.SemaphoreType.DMA,
        ],
    )
    def body(t_hbm, cf_hbm, out_hbm, coord_v, idx_v, wgt_v, rows_v, outb_v,
             sem_c, sem_g, sem_o):
        wid = lax.axis_index("s") * _NC + lax.axis_index("c")
        base_w = wid * PW
        iota = lax.iota(jnp.int32, _L)

        def coord_copies(b, p):
            base = base_w + b * K
            return [
                pltpu.make_async_copy(
                    cf_hbm.at[pl.ds(ch * N + base, K)],
                    coord_v.at[p, ch], sem_c)
                for ch in range(3)
            ]

        def gather_copies(p):
            return [
                pltpu.make_async_copy(
                    t_hbm.at[idx_v.at[p, pl.ds(j * CH, CH)]],
                    rows_v.at[p, pl.ds(j * CH, CH)],
                    sem_g,
                )
                for j in range(NCH)
            ]

        def out_copies(b, p):
            base = base_w + b * K
            return [
                pltpu.make_async_copy(
                    outb_v.at[p, ch],
                    out_hbm.at[pl.ds(ch * N + base, K)], sem_o)
                for ch in range(C)
            ]

        def stage(b, p):
            for cpy in coord_copies(b, p):
                cpy.wait()

            @pl.loop(0, G, unroll=4)
            def _idx(g):
                s = g * _L
                z = coord_v[p, 0, pl.ds(s, _L)]
                y = coord_v[p, 1, pl.ds(s, _L)]
                x = coord_v[p, 2, pl.ds(s, _L)]
                z = jnp.minimum(jnp.maximum(z, 0.0), D - 1.0)
                y = jnp.minimum(jnp.maximum(y, 0.0), H - 1.0)
                x = jnp.minimum(jnp.maximum(x, 0.0), W - 1.0)
                zi = z.astype(jnp.int32)
                yi = y.astype(jnp.int32)
                xi = x.astype(jnp.int32)
                wgt_v[p, 0, pl.ds(s, _L)] = z - zi.astype(jnp.float32)
                wgt_v[p, 1, pl.ds(s, _L)] = y - yi.astype(jnp.float32)
                wgt_v[p, 2, pl.ds(s, _L)] = x - xi.astype(jnp.float32)
                idx_v[p, pl.ds(s, _L)] = (zi * H + yi) * W + xi

            for cpy in gather_copies(p):
                cpy.start()

        for cpy in coord_copies(0, 0):
            cpy.start()
        stage(0, 0)

        @pl.loop(0, NB, step=2)
        def _blk2(b0):
            for p in range(2):
                b = b0 + p

                @pl.when(b + 1 < NB)
                def _():
                    for cpy in coord_copies(b + 1, 1 - p):
                        cpy.start()
                    stage(b + 1, 1 - p)

                for cpy in gather_copies(p):
                    cpy.wait()

                @pl.when(b >= 2)
                def _():
                    for cpy in out_copies(b - 2, p):
                        cpy.wait()

                @pl.loop(0, G, unroll=4)
                def _interp(g):
                    s = g * _L
                    wz = wgt_v[p, 0, pl.ds(s, _L)]
                    wy = wgt_v[p, 1, pl.ds(s, _L)]
                    wx = wgt_v[p, 2, pl.ds(s, _L)]
                    uz = 1.0 - wz
                    uy = 1.0 - wy
                    ux = 1.0 - wx
                    w00 = uy * ux
                    w01 = uy * wx
                    w10 = wy * ux
                    w11 = wy * wx
                    ridx = iota + s

                    def col(j):
                        cidx = jnp.full((_L,), j, jnp.int32)
                        return plsc.load_gather(rows_v.at[p], [ridx, cidx])

                    a = [col(j) for j in range(16)]
                    out0 = (uz * (a[0] * w00 + a[1] * w01
                                  + a[2] * w10 + a[3] * w11)
                            + wz * (a[4] * w00 + a[5] * w01
                                    + a[6] * w10 + a[7] * w11))
                    out1 = (uz * (a[8] * w00 + a[9] * w01
                                  + a[10] * w10 + a[11] * w11)
                            + wz * (a[12] * w00 + a[13] * w01
                                    + a[14] * w10 + a[15] * w11))
                    outb_v[p, 0, pl.ds(s, _L)] = out0
                    outb_v[p, 1, pl.ds(s, _L)] = out1

                for cpy in out_copies(b, p):
                    cpy.start()

        for cpy in out_copies(NB - 2, 0):
            cpy.wait()
        for cpy in out_copies(NB - 1, 1):
            cpy.wait()

    return body


def kernel(values, coordinates):
    B, C, D, H, W = values.shape
    assert B == 1 and C == 2
    N = D * H * W
    HW = H * W
    vf = jnp.pad(values.reshape(C * N), (0, HW + W + 8))
    table = _builder(C, N, HW, W, 2048)(vf)
    cf = coordinates.reshape(3 * N)
    out = _sampler(D, H, W, C, N, 2048)(table, cf)
    return out.reshape(B, C, D, H, W)


# speedup vs baseline: 1.0384x; 1.0384x over previous
import functools

import jax
import jax.numpy as jnp
from jax import lax
from jax.experimental import pallas as pl
from jax.experimental.pallas import tpu as pltpu
from jax.experimental.pallas import tpu_sc as plsc

_NC, _NS, _L = 2, 16, 16
_NW = _NC * _NS

_SC_PARAMS = pltpu.CompilerParams(
    needs_layout_passes=False, use_tc_tiling_on_sc=False)


def _mesh():
    return plsc.VectorSubcoreMesh(core_axis_name="c", subcore_axis_name="s")


@functools.cache
def _builder(C, N, HW, W, KB):
    PW = N // _NW
    NB = PW // KB
    G = KB // _L
    SPAN = KB + W + 8

    @functools.partial(
        pl.kernel,
        out_type=jax.ShapeDtypeStruct((N * 16,), jnp.float32),
        mesh=_mesh(),
        compiler_params=_SC_PARAMS,
        scratch_types=[
            pltpu.VMEM((2, 4, SPAN), jnp.float32),
            pltpu.VMEM((2, KB * _L), jnp.float32),
            pltpu.SemaphoreType.DMA,
            pltpu.SemaphoreType.DMA,
        ],
    )
    def body(vf_hbm, t_hbm, streams_v, rows_v, sem_i, sem_o):
        wid = lax.axis_index("s") * _NC + lax.axis_index("c")
        base_w = wid * PW
        iota16 = lax.iota(jnp.int32, _L) * _L

        def in_copies(b, p):
            base = base_w + b * KB
            return [
                pltpu.make_async_copy(
                    vf_hbm.at[pl.ds(c * N + base + zo, SPAN)],
                    streams_v.at[p, c * 2 + zi],
                    sem_i,
                )
                for c in range(C)
                for zi, zo in enumerate((0, HW))
            ]

        def out_copy(b, p):
            base = base_w + b * KB
            return pltpu.make_async_copy(
                rows_v.at[p], t_hbm.at[pl.ds(base * 16, KB * 16)], sem_o)

        for cpy in in_copies(0, 0):
            cpy.start()

        @pl.loop(0, NB, step=2)
        def _blk2(b0):
            for p in range(2):
                b = b0 + p
                for cpy in in_copies(b, p):
                    cpy.wait()

                @pl.when(b + 1 < NB)
                def _():
                    for cpy in in_copies(b + 1, 1 - p):
                        cpy.start()

                @pl.when(b >= 2)
                def _():
                    out_copy(b - 2, p).wait()

                @pl.loop(0, G)
                def _ilv(g):
                    s = g * _L
                    fidx = iota16 + s * _L
                    for c in range(C):
                        for zi in range(2):
                            for yx, yxo in enumerate((0, 1, W, W + 1)):
                                col = c * 8 + zi * 4 + yx
                                vec = streams_v[p, c * 2 + zi,
                                                pl.ds(s + yxo, _L)]
                                plsc.store_scatter(
                                    rows_v.at[p], [fidx + col], vec)

                out_copy(b, p).start()

        out_copy(NB - 2, 0).wait()
        out_copy(NB - 1, 1).wait()

    return body


@functools.cache
def _sampler(D, H, W, C, N, K):
    PW = N // _NW
    NB = PW // K
    G = K // _L
    CH = 128
    NCH = K // CH

    @functools.partial(
        pl.kernel,
        out_type=jax.ShapeDtypeStruct((C * N,), jnp.float32),
        mesh=_mesh(),
        compiler_params=_SC_PARAMS,
        scratch_types=[
            pltpu.VMEM((2, 3, K), jnp.float32),
            pltpu.VMEM((2, K), jnp.int32),
            pltpu.VMEM((2, 3, K), jnp.float32),
            pltpu.VMEM((2, K, _L), jnp.float32),
            pltpu.VMEM((2, C, K), jnp.float32),
            pltpu.SemaphoreType.DMA,
            pltpu.SemaphoreType.DMA,
            pltpu.SemaphoreType.DMA,
        ],
    )
    def body(t_hbm, cf_hbm, out_hbm, coord_v, idx_v, wgt_v, rows_v, outb_v,
             sem_c, sem_g, sem_o):
        wid = lax.axis_index("s") * _NC + lax.axis_index("c")
        base_w = wid * PW
        iota = lax.iota(jnp.int32, _L)

        def coord_copies(b, p):
            base = base_w + b * K
            return [
                pltpu.make_async_copy(
                    cf_hbm.at[pl.ds(ch * N + base, K)],
                    coord_v.at[p, ch], sem_c)
                for ch in range(3)
            ]

        def gather_copies(p):
            return [
                pltpu.make_async_copy(
                    t_hbm.at[idx_v.at[p, pl.ds(j * CH, CH)]],
                    rows_v.at[p, pl.ds(j * CH, CH)],
                    sem_g,
                )
                for j in range(NCH)
            ]

        def out_copies(b, p):
            base = base_w + b * K
            return [
                pltpu.make_async_copy(
                    outb_v.at[p, ch],
                    out_hbm.at[pl.ds(ch * N + base, K)], sem_o)
                for ch in range(C)
            ]

        def stage(b, p):
            for cpy in coord_copies(b, p):
                cpy.wait()

            @pl.loop(0, G)
            def _idx(g):
                s = g * _L
                z = coord_v[p, 0, pl.ds(s, _L)]
                y = coord_v[p, 1, pl.ds(s, _L)]
                x = coord_v[p, 2, pl.ds(s, _L)]
                z = jnp.minimum(jnp.maximum(z, 0.0), D - 1.0)
                y = jnp.minimum(jnp.maximum(y, 0.0), H - 1.0)
                x = jnp.minimum(jnp.maximum(x, 0.0), W - 1.0)
                zi = z.astype(jnp.int32)
                yi = y.astype(jnp.int32)
                xi = x.astype(jnp.int32)
                wgt_v[p, 0, pl.ds(s, _L)] = z - zi.astype(jnp.float32)
                wgt_v[p, 1, pl.ds(s, _L)] = y - yi.astype(jnp.float32)
                wgt_v[p, 2, pl.ds(s, _L)] = x - xi.astype(jnp.float32)
                idx_v[p, pl.ds(s, _L)] = (zi * H + yi) * W + xi

            for cpy in gather_copies(p):
                cpy.start()

        for cpy in coord_copies(0, 0):
            cpy.start()
        stage(0, 0)

        @pl.loop(0, NB, step=2)
        def _blk2(b0):
            for p in range(2):
                b = b0 + p

                @pl.when(b + 1 < NB)
                def _():
                    for cpy in coord_copies(b + 1, 1 - p):
                        cpy.start()
                    stage(b + 1, 1 - p)

                for cpy in gather_copies(p):
                    cpy.wait()

                @pl.when(b >= 2)
                def _():
                    for cpy in out_copies(b - 2, p):
                        cpy.wait()

                @pl.loop(0, G)
                def _interp(g):
                    s = g * _L
                    wz = wgt_v[p, 0, pl.ds(s, _L)]
                    wy = wgt_v[p, 1, pl.ds(s, _L)]
                    wx = wgt_v[p, 2, pl.ds(s, _L)]
                    uz = 1.0 - wz
                    uy = 1.0 - wy
                    ux = 1.0 - wx
                    w00 = uy * ux
                    w01 = uy * wx
                    w10 = wy * ux
                    w11 = wy * wx
                    ridx = iota + s

                    def col(j):
                        cidx = jnp.full((_L,), j, jnp.int32)
                        return plsc.load_gather(rows_v.at[p], [ridx, cidx])

                    a = [col(j) for j in range(16)]
                    out0 = (uz * (a[0] * w00 + a[1] * w01
                                  + a[2] * w10 + a[3] * w11)
                            + wz * (a[4] * w00 + a[5] * w01
                                    + a[6] * w10 + a[7] * w11))
                    out1 = (uz * (a[8] * w00 + a[9] * w01
                                  + a[10] * w10 + a[11] * w11)
                            + wz * (a[12] * w00 + a[13] * w01
                                    + a[14] * w10 + a[15] * w11))
                    outb_v[p, 0, pl.ds(s, _L)] = out0
                    outb_v[p, 1, pl.ds(s, _L)] = out1

                for cpy in out_copies(b, p):
                    cpy.start()

        for cpy in out_copies(NB - 2, 0):
            cpy.wait()
        for cpy in out_copies(NB - 1, 1):
            cpy.wait()

    return body


def kernel(values, coordinates):
    B, C, D, H, W = values.shape
    assert B == 1 and C == 2
    N = D * H * W
    HW = H * W
    vf = jnp.pad(values.reshape(C * N), (0, HW + W + 8))
    table = _builder(C, N, HW, W, 2048)(vf).reshape(N, 16)
    cf = coordinates.reshape(3 * N)
    out = _sampler(D, H, W, C, N, 2048)(table, cf)
    return out.reshape(B, C, D, H, W)


# speedup vs baseline: 1.0416x; 1.0031x over previous
import functools

import jax
import jax.numpy as jnp
from jax import lax
from jax.experimental import pallas as pl
from jax.experimental.pallas import tpu as pltpu
from jax.experimental.pallas import tpu_sc as plsc

_NC, _NS, _L = 2, 16, 16
_NW = _NC * _NS

_SC_PARAMS = pltpu.CompilerParams(
    needs_layout_passes=False, use_tc_tiling_on_sc=False)


def _mesh():
    return plsc.VectorSubcoreMesh(core_axis_name="c", subcore_axis_name="s")


@functools.cache
def _builder(C, N, HW, W, KB):
    PW = N // _NW
    NB = PW // KB
    G = KB // _L
    SPAN = KB + W + 8

    @functools.partial(
        pl.kernel,
        out_type=jax.ShapeDtypeStruct((N * 16,), jnp.float32),
        mesh=_mesh(),
        compiler_params=_SC_PARAMS,
        scratch_types=[
            pltpu.VMEM((2, 4, SPAN), jnp.float32),
            pltpu.VMEM((2, KB * _L), jnp.float32),
            pltpu.SemaphoreType.DMA,
            pltpu.SemaphoreType.DMA,
        ],
    )
    def body(vf_hbm, t_hbm, streams_v, rows_v, sem_i, sem_o):
        wid = lax.axis_index("s") * _NC + lax.axis_index("c")
        base_w = wid * PW
        iota16 = lax.iota(jnp.int32, _L) * _L

        def in_copies(b, p):
            base = base_w + b * KB
            return [
                pltpu.make_async_copy(
                    vf_hbm.at[pl.ds(c * N + base + zo, SPAN)],
                    streams_v.at[p, c * 2 + zi],
                    sem_i,
                )
                for c in range(C)
                for zi, zo in enumerate((0, HW))
            ]

        def out_copy(b, p):
            base = base_w + b * KB
            return pltpu.make_async_copy(
                rows_v.at[p], t_hbm.at[pl.ds(base * 16, KB * 16)], sem_o)

        for cpy in in_copies(0, 0):
            cpy.start()

        @pl.loop(0, NB, step=2)
        def _blk2(b0):
            for p in range(2):
                b = b0 + p
                for cpy in in_copies(b, p):
                    cpy.wait()

                @pl.when(b + 1 < NB)
                def _():
                    for cpy in in_copies(b + 1, 1 - p):
                        cpy.start()

                @pl.when(b >= 2)
                def _():
                    out_copy(b - 2, p).wait()

                @pl.loop(0, G, unroll=2)
                def _ilv(g):
                    s = g * _L
                    fidx = iota16 + s * _L
                    for c in range(C):
                        for zi in range(2):
                            for yx, yxo in enumerate((0, 1, W, W + 1)):
                                col = c * 8 + zi * 4 + yx
                                vec = streams_v[p, c * 2 + zi,
                                                pl.ds(s + yxo, _L)]
                                plsc.store_scatter(
                                    rows_v.at[p], [fidx + col], vec)

                out_copy(b, p).start()

        out_copy(NB - 2, 0).wait()
        out_copy(NB - 1, 1).wait()

    return body


@functools.cache
def _sampler(D, H, W, C, N, K):
    PW = N // _NW
    NB = PW // K
    G = K // _L
    CH = 128
    NCH = K // CH

    @functools.partial(
        pl.kernel,
        out_type=jax.ShapeDtypeStruct((C * N,), jnp.float32),
        mesh=_mesh(),
        compiler_params=_SC_PARAMS,
        scratch_types=[
            pltpu.VMEM((2, 3, K), jnp.float32),
            pltpu.VMEM((2, K), jnp.int32),
            pltpu.VMEM((2, 3, K), jnp.float32),
            pltpu.VMEM((2, K, _L), jnp.float32),
            pltpu.VMEM((2, C, K), jnp.float32),
            pltpu.SemaphoreType.DMA,
            pltpu.SemaphoreType.DMA,
            pltpu.SemaphoreType.DMA,
        ],
    )
    def body(t_hbm, cf_hbm, out_hbm, coord_v, idx_v, wgt_v, rows_v, outb_v,
             sem_c, sem_g, sem_o):
        wid = lax.axis_index("s") * _NC + lax.axis_index("c")
        base_w = wid * PW
        iota = lax.iota(jnp.int32, _L)

        def coord_copies(b, p):
            base = base_w + b * K
            return [
                pltpu.make_async_copy(
                    cf_hbm.at[pl.ds(ch * N + base, K)],
                    coord_v.at[p, ch], sem_c)
                for ch in range(3)
            ]

        def gather_copies(p):
            return [
                pltpu.make_async_copy(
                    t_hbm.at[idx_v.at[p, pl.ds(j * CH, CH)]],
                    rows_v.at[p, pl.ds(j * CH, CH)],
                    sem_g,
                )
                for j in range(NCH)
            ]

        def out_copies(b, p):
            base = base_w + b * K
            return [
                pltpu.make_async_copy(
                    outb_v.at[p, ch],
                    out_hbm.at[pl.ds(ch * N + base, K)], sem_o)
                for ch in range(C)
            ]

        def stage(b, p):
            for cpy in coord_copies(b, p):
                cpy.wait()

            @pl.loop(0, G)
            def _idx(g):
                s = g * _L
                z = coord_v[p, 0, pl.ds(s, _L)]
                y = coord_v[p, 1, pl.ds(s, _L)]
                x = coord_v[p, 2, pl.ds(s, _L)]
                z = jnp.minimum(jnp.maximum(z, 0.0), D - 1.0)
                y = jnp.minimum(jnp.maximum(y, 0.0), H - 1.0)
                x = jnp.minimum(jnp.maximum(x, 0.0), W - 1.0)
                zi = z.astype(jnp.int32)
                yi = y.astype(jnp.int32)
                xi = x.astype(jnp.int32)
                wgt_v[p, 0, pl.ds(s, _L)] = z - zi.astype(jnp.float32)
                wgt_v[p, 1, pl.ds(s, _L)] = y - yi.astype(jnp.float32)
                wgt_v[p, 2, pl.ds(s, _L)] = x - xi.astype(jnp.float32)
                idx_v[p, pl.ds(s, _L)] = (zi * H + yi) * W + xi

            for cpy in gather_copies(p):
                cpy.start()

        for cpy in coord_copies(0, 0):
            cpy.start()
        stage(0, 0)

        @pl.loop(0, NB, step=2)
        def _blk2(b0):
            for p in range(2):
                b = b0 + p

                @pl.when(b + 1 < NB)
                def _():
                    for cpy in coord_copies(b + 1, 1 - p):
                        cpy.start()
                    stage(b + 1, 1 - p)

                for cpy in gather_copies(p):
                    cpy.wait()

                @pl.when(b >= 2)
                def _():
                    for cpy in out_copies(b - 2, p):
                        cpy.wait()

                @pl.loop(0, G)
                def _interp(g):
                    s = g * _L
                    wz = wgt_v[p, 0, pl.ds(s, _L)]
                    wy = wgt_v[p, 1, pl.ds(s, _L)]
                    wx = wgt_v[p, 2, pl.ds(s, _L)]
                    uz = 1.0 - wz
                    uy = 1.0 - wy
                    ux = 1.0 - wx
                    w00 = uy * ux
                    w01 = uy * wx
                    w10 = wy * ux
                    w11 = wy * wx
                    ridx = iota + s

                    def col(j):
                        cidx = jnp.full((_L,), j, jnp.int32)
                        return plsc.load_gather(rows_v.at[p], [ridx, cidx])

                    a = [col(j) for j in range(16)]
                    out0 = (uz * (a[0] * w00 + a[1] * w01
                                  + a[2] * w10 + a[3] * w11)
                            + wz * (a[4] * w00 + a[5] * w01
                                    + a[6] * w10 + a[7] * w11))
                    out1 = (uz * (a[8] * w00 + a[9] * w01
                                  + a[10] * w10 + a[11] * w11)
                            + wz * (a[12] * w00 + a[13] * w01
                                    + a[14] * w10 + a[15] * w11))
                    outb_v[p, 0, pl.ds(s, _L)] = out0
                    outb_v[p, 1, pl.ds(s, _L)] = out1

                for cpy in out_copies(b, p):
                    cpy.start()

        for cpy in out_copies(NB - 2, 0):
            cpy.wait()
        for cpy in out_copies(NB - 1, 1):
            cpy.wait()

    return body


def kernel(values, coordinates):
    B, C, D, H, W = values.shape
    assert B == 1 and C == 2
    N = D * H * W
    HW = H * W
    vf = jnp.pad(values.reshape(C * N), (0, HW + W + 8))
    table = _builder(C, N, HW, W, 2048)(vf).reshape(N, 16)
    cf = coordinates.reshape(3 * N)
    out = _sampler(D, H, W, C, N, 2048)(table, cf)
    return out.reshape(B, C, D, H, W)
